# baseline (speedup 1.0000x reference)
import functools

import jax
import jax.numpy as jnp
from jax import lax
from jax.experimental import pallas as pl
from jax.experimental.pallas import tpu as pltpu
from jax.experimental.pallas import tpu_sc as plsc

_BATCH = 16384
_ROWS = 1000000
_NW = 32
_NPW = _BATCH // _NW
_NIDX = _NPW * 32
_NCH = 128
_CSZ = _NIDX // _NCH


def _body(xt_hbm, tbl_hbm, wp_hbm, out_hbm,
          users_v, lb_v, wp_v, idx_v, val_v, o0_v, o1_v, sem):
    wid = lax.axis_index("s") * 2 + lax.axis_index("c")
    base = wid * _NPW

    pltpu.sync_copy(xt_hbm.at[pl.ds(base, _NPW)], users_v.at[pl.ds(0, _NPW)])
    pltpu.sync_copy(xt_hbm.at[pl.ds(_BATCH + base, _NPW)],
                    users_v.at[pl.ds(_NPW, _NPW)])
    pltpu.sync_copy(xt_hbm.at[pl.ds(4 * _BATCH + base, _NPW)], lb_v)
    pltpu.sync_copy(wp_hbm, wp_v)

    def build(g, carry):
        blk = g & 3
        c = (g >> 2) & 15
        sel = g >> 6
        src = sel * _NPW + blk * 128
        cbase = c * _ROWS
        for t in range(8):
            u = users_v[pl.ds(src + t * 16, 16)].astype(jnp.int32)
            idx_v[pl.ds(g * _CSZ + t * 16, 16)] = u + cbase
        pltpu.async_copy(
            tbl_hbm.at[idx_v.at[pl.ds(g * _CSZ, _CSZ)]],
            val_v.at[pl.ds(g * _CSZ, _CSZ)], sem)
        return carry

    lax.fori_loop(0, _NCH, build, 0)
    pltpu.make_async_copy(tbl_hbm.at[pl.ds(0, _NIDX)], val_v, sem).wait()

    wlbl = wp_v[pl.ds(32 * 16, 16)]
    wdb = wp_v[pl.ds(33 * 16, 16)]
    wv = [wp_v[pl.ds(c * 16, 16)] for c in range(32)]

    def compute_blk(blk, carry):
        for t in range(8):
            e = blk * 128 + t * 16
            acc = lb_v[pl.ds(e, 16)] * wlbl + wdb
            for c in range(16):
                vl = val_v[pl.ds((c * 4 + blk) * 128 + t * 16, 16)]
                acc = acc + vl * wv[c]
            for c in range(16):
                vc = val_v[pl.ds((64 + c * 4 + blk) * 128 + t * 16, 16)]
                acc = acc + vc * wv[16 + c]
            e0 = jnp.exp(jnp.minimum(-acc, 0.0))
            e1 = jnp.exp(jnp.minimum(acc, 0.0))
            rz = 1.0 / (e0 + e1)
            o0_v[pl.ds(e, 16)] = e0 * rz
            o1_v[pl.ds(e, 16)] = e1 * rz
        return carry

    lax.fori_loop(0, 4, compute_blk, 0)

    pltpu.sync_copy(o0_v, out_hbm.at[pl.ds(base, _NPW)])
    pltpu.sync_copy(o1_v, out_hbm.at[pl.ds(_BATCH + base, _NPW)])


@functools.partial(jax.jit, static_argnums=())
def _run(xt_flat, tbl_flat, wp):
    mesh = plsc.VectorSubcoreMesh(core_axis_name="c", subcore_axis_name="s")
    f = pl.kernel(
        _body,
        out_type=jax.ShapeDtypeStruct((2 * _BATCH,), jnp.float32),
        mesh=mesh,
        scratch_types=[
            pltpu.VMEM((2 * _NPW,), jnp.float32),
            pltpu.VMEM((_NPW,), jnp.float32),
            pltpu.VMEM((34 * 16,), jnp.float32),
            pltpu.VMEM((_NIDX,), jnp.int32),
            pltpu.VMEM((_NIDX,), jnp.float32),
            pltpu.VMEM((_NPW,), jnp.float32),
            pltpu.VMEM((_NPW,), jnp.float32),
            pltpu.SemaphoreType.DMA,
        ],
        compiler_params=pltpu.CompilerParams(
            needs_layout_passes=False, use_tc_tiling_on_sc=False),
    )
    return f(xt_flat, tbl_flat, wp)


def kernel(x, table, W, b):
    wd = W[1] - W[0]
    wp = jnp.concatenate([
        jnp.broadcast_to(wd[:32, None], (32, 16)),
        jnp.full((1, 16), wd[32], jnp.float32),
        jnp.full((1, 16), b[1] - b[0], jnp.float32),
    ], axis=0).reshape(-1)
    xt = x.transpose(2, 1, 0).reshape(-1)
    tbl = table.transpose(1, 0).reshape(-1)
    out = _run(xt, tbl, wp)
    return out.reshape(2, _BATCH).transpose(1, 0)


# speedup vs baseline: 1.4783x; 1.4783x over previous
import functools

import jax
import jax.numpy as jnp
from jax import lax
from jax.experimental import pallas as pl
from jax.experimental.pallas import tpu as pltpu
from jax.experimental.pallas import tpu_sc as plsc

_BATCH = 16384
_ROWS = 1000000
_NW = 32
_NPW = _BATCH // _NW
_CSZ = 128
_NBLK = _NPW // _CSZ


def _body(xt_hbm, tbl_hbm, wp_hbm, out_hbm,
          users_v, lb_v, wp_v, idx_v, val_v, o0_v, o1_v, sem):
    wid = lax.axis_index("s") * 2 + lax.axis_index("c")
    base = wid * _NPW

    pltpu.sync_copy(xt_hbm.at[pl.ds(base, _NPW)], users_v.at[pl.ds(0, _NPW)])
    pltpu.sync_copy(xt_hbm.at[pl.ds(_BATCH + base, _NPW)],
                    users_v.at[pl.ds(_NPW, _NPW)])
    pltpu.sync_copy(xt_hbm.at[pl.ds(4 * _BATCH + base, _NPW)], lb_v)
    pltpu.sync_copy(wp_hbm, wp_v)

    def build(g, carry):
        for t in range(8):
            u = users_v[pl.ds(g * _CSZ + t * 16, 16)].astype(jnp.int32)
            idx_v[pl.ds(g * _CSZ + t * 16, 16)] = u
        return carry

    lax.fori_loop(0, 2 * _NBLK, build, 0)

    for sel in range(2):
        for blk in range(_NBLK):
            isl = idx_v.at[pl.ds((sel * _NBLK + blk) * _CSZ, _CSZ)]
            for c in range(16):
                pltpu.async_copy(
                    tbl_hbm.at[c].at[isl],
                    val_v.at[pl.ds(((sel * 16 + c) * _NBLK + blk) * _CSZ,
                                   _CSZ)],
                    sem)
    pltpu.make_async_copy(xt_hbm.at[pl.ds(0, 32 * _NPW)], val_v, sem).wait()

    wlbl = wp_v[pl.ds(32 * 16, 16)]
    wdb = wp_v[pl.ds(33 * 16, 16)]
    wv = [wp_v[pl.ds(c * 16, 16)] for c in range(32)]

    def compute_blk(blk, carry):
        for t in range(8):
            e = blk * 128 + t * 16
            acc = lb_v[pl.ds(e, 16)] * wlbl + wdb
            for c in range(16):
                vl = val_v[pl.ds((c * _NBLK + blk) * _CSZ + t * 16, 16)]
                acc = acc + vl * wv[c]
            for c in range(16):
                vc = val_v[pl.ds(((16 + c) * _NBLK + blk) * _CSZ + t * 16, 16)]
                acc = acc + vc * wv[16 + c]
            e0 = jnp.exp(jnp.minimum(-acc, 0.0))
            e1 = jnp.exp(jnp.minimum(acc, 0.0))
            rz = 1.0 / (e0 + e1)
            o0_v[pl.ds(e, 16)] = e0 * rz
            o1_v[pl.ds(e, 16)] = e1 * rz
        return carry

    lax.fori_loop(0, _NBLK, compute_blk, 0)

    pltpu.sync_copy(o0_v, out_hbm.at[pl.ds(base, _NPW)])
    pltpu.sync_copy(o1_v, out_hbm.at[pl.ds(_BATCH + base, _NPW)])


@functools.partial(jax.jit, static_argnums=())
def _run(xt_flat, tbl_t, wp):
    mesh = plsc.VectorSubcoreMesh(core_axis_name="c", subcore_axis_name="s")
    f = pl.kernel(
        _body,
        out_type=jax.ShapeDtypeStruct((2 * _BATCH,), jnp.float32),
        mesh=mesh,
        scratch_types=[
            pltpu.VMEM((2 * _NPW,), jnp.float32),
            pltpu.VMEM((_NPW,), jnp.float32),
            pltpu.VMEM((34 * 16,), jnp.float32),
            pltpu.VMEM((2 * _NPW,), jnp.int32),
            pltpu.VMEM((32 * _NPW,), jnp.float32),
            pltpu.VMEM((_NPW,), jnp.float32),
            pltpu.VMEM((_NPW,), jnp.float32),
            pltpu.SemaphoreType.DMA,
        ],
        compiler_params=pltpu.CompilerParams(
            needs_layout_passes=False, use_tc_tiling_on_sc=False),
    )
    return f(xt_flat, tbl_t, wp)


def kernel(x, table, W, b):
    wd = W[1] - W[0]
    wp = jnp.concatenate([
        jnp.broadcast_to(wd[:32, None], (32, 16)),
        jnp.full((1, 16), wd[32], jnp.float32),
        jnp.full((1, 16), b[1] - b[0], jnp.float32),
    ], axis=0).reshape(-1)
    xt = x.transpose(2, 1, 0).reshape(-1)
    tbl_lin = jnp.concatenate([table[:, c] for c in range(16)])
    out = _run(xt, tbl_lin.reshape(16, _ROWS), wp)
    return out.reshape(2, _BATCH).transpose(1, 0)
